# baseline (speedup 1.0000x reference)
import functools

import jax
import jax.numpy as jnp
from jax.experimental import pallas as pl
from jax.experimental.pallas import tpu as pltpu

D = 128


def _matmul_body(x_ref, w_ref, b_ref, o_ref):
    o_ref[...] = (
        jnp.dot(x_ref[...], w_ref[...], preferred_element_type=jnp.float32)
        + b_ref[...]
    )


def _matmul(x, w_t, b, bm=512):
    m, k = x.shape
    nout = w_t.shape[1]
    assert m % bm == 0, (m, bm)
    return pl.pallas_call(
        _matmul_body,
        grid=(m // bm,),
        in_specs=[
            pl.BlockSpec((bm, k), lambda i: (i, 0)),
            pl.BlockSpec((k, nout), lambda i: (0, 0)),
            pl.BlockSpec((nout,), lambda i: (0,)),
        ],
        out_specs=pl.BlockSpec((bm, nout), lambda i: (i, 0)),
        out_shape=jax.ShapeDtypeStruct((m, nout), jnp.float32),
    )(x, w_t, b)


def _ln_silu(t, g, b):
    mu = jnp.mean(t, axis=-1, keepdims=True)
    var = jnp.mean((t - mu) ** 2, axis=-1, keepdims=True)
    ln = (t - mu) / jnp.sqrt(var + 1e-5) * g + b
    return ln * jax.nn.sigmoid(ln)


def _edge_stage_body(m_ref, bh_ref, y_ref, g_ref, b_ref,
                     sig_ref, sbh_ref, ymid_ref):
    m = m_ref[...]
    sig = jax.nn.sigmoid(m)
    sig_ref[...] = sig
    sbh_ref[...] = sig * bh_ref[...]
    ymid_ref[...] = y_ref[...] + _ln_silu(m, g_ref[...], b_ref[...])


def _edge_stage(m_pre, bh_src, y, g, b, bm=640):
    e = m_pre.shape[0]
    assert e % bm == 0
    out_sd = jax.ShapeDtypeStruct((e, D), jnp.float32)
    return pl.pallas_call(
        _edge_stage_body,
        grid=(e // bm,),
        in_specs=[
            pl.BlockSpec((bm, D), lambda i: (i, 0)),
            pl.BlockSpec((bm, D), lambda i: (i, 0)),
            pl.BlockSpec((bm, D), lambda i: (i, 0)),
            pl.BlockSpec((D,), lambda i: (0,)),
            pl.BlockSpec((D,), lambda i: (0,)),
        ],
        out_specs=[pl.BlockSpec((bm, D), lambda i: (i, 0))] * 3,
        out_shape=[out_sd, out_sd, out_sd],
    )(m_pre, bh_src, y, g, b)


def _node_fin_body(ax_ref, ssh_ref, ss_ref, x_ref, g_ref, b_ref, o_ref):
    h = ssh_ref[...] / (ss_ref[...] + 1e-6)
    t = ax_ref[...] + h
    o_ref[...] = x_ref[...] + _ln_silu(t, g_ref[...], b_ref[...])


def _node_fin(ax, ssh, ss, x, g, b, bm=400):
    n = x.shape[0]
    assert n % bm == 0
    return pl.pallas_call(
        _node_fin_body,
        grid=(n // bm,),
        in_specs=[
            pl.BlockSpec((bm, D), lambda i: (i, 0)),
            pl.BlockSpec((bm, D), lambda i: (i, 0)),
            pl.BlockSpec((bm, D), lambda i: (i, 0)),
            pl.BlockSpec((bm, D), lambda i: (i, 0)),
            pl.BlockSpec((D,), lambda i: (0,)),
            pl.BlockSpec((D,), lambda i: (0,)),
        ],
        out_specs=pl.BlockSpec((bm, D), lambda i: (i, 0)),
        out_shape=jax.ShapeDtypeStruct((n, D), jnp.float32),
    )(ax, ssh, ss, x, g, b)


def _egc_layer(node_feats, edge_feats, src, dst, p, bm_nodes):
    n = node_feats.shape[0]
    wn = jnp.concatenate(
        [p['W_src_gate'].T, p['W_dst_gate'].T,
         p['W_dst_update'].T, p['W_src_update'].T], axis=1)
    bn = jnp.concatenate(
        [p['b_src_gate'], p['b_dst_gate'],
         p['b_dst_update'], p['b_src_update']], axis=0)
    nodes4 = _matmul(node_feats, wn, bn, bm=bm_nodes)
    e_src_t = nodes4[:, 0 * D:1 * D]
    e_dst_t = nodes4[:, 1 * D:2 * D]
    bh_t = nodes4[:, 2 * D:3 * D]
    ax_t = nodes4[:, 3 * D:4 * D]

    ey = _matmul(edge_feats, p['W_edge_gate'].T, p['b_edge_gate'], bm=640)

    m_pre = e_src_t[src] + e_dst_t[dst] + ey
    bh_src = bh_t[src]

    sigma, sbh, y_mid = _edge_stage(m_pre, bh_src, edge_feats,
                                    p['g_edges'], p['b_edges'])

    ssh = jax.ops.segment_sum(sbh, dst, num_segments=n)
    ss = jax.ops.segment_sum(sigma, dst, num_segments=n)

    x_out = _node_fin(ax_t, ssh, ss, node_feats,
                      p['g_nodes'], p['b_nodes'], bm=bm_nodes)
    return x_out, y_mid


def kernel(x, y, z, nu_params, eu_params, edge_index, lg_edge_index):
    src, dst = edge_index[0], edge_index[1]
    x_out, m = _egc_layer(x, y, src, dst, nu_params, bm_nodes=400)
    lsrc, ldst = lg_edge_index[0], lg_edge_index[1]
    y_out, z_out = _egc_layer(m, z, lsrc, ldst, eu_params, bm_nodes=640)
    return (x_out, y_out, z_out)
